# baseline (speedup 1.0000x reference)
import jax
import jax.numpy as jnp
from jax import lax
from jax.experimental import pallas as pl
from jax.experimental.pallas import tpu as pltpu
from jax.experimental.pallas import tpu_sc as plsc

N = 10000
NP = 10112
E = 320000
D = 128
BN_EPS = 1e-5

NC = 2
NS = 16
CHUNK = 128
FAST2 = 1
ROUND = 53
E_FAST = NS * 2 * ROUND * CHUNK
E_SLOW = NS * 1 * ROUND * CHUNK
E_PAD = E_FAST + E_SLOW
TRASH_ROW = N
ROWS_PER_TILE = NP // NS


def _dense_body(x_ref, g_ref, b_ref, wt_ref, h_ref, hs_ref):
    x = x_ref[...]
    mean = jnp.mean(x, axis=0, keepdims=True)
    xc = x - mean
    var = jnp.mean(xc * xc, axis=0, keepdims=True)
    inv = lax.rsqrt(var + BN_EPS)
    xh = jnp.maximum(xc * (inv * g_ref[...]) + b_ref[...], 0.0)
    h = jnp.dot(xh, wt_ref[...], preferred_element_type=jnp.float32)
    h_ref[...] = h
    hs_ref[0:N, :] = 0.5 * h * h


_dense_call = pl.pallas_call(
    _dense_body,
    out_shape=(
        jax.ShapeDtypeStruct((N, D), jnp.float32),
        jax.ShapeDtypeStruct((NP, D), jnp.float32),
    ),
)


def _sc_body(h_hbm, eidx_hbm, hs_hbm, out_hbm, cidx, rows, acc, gsem):
    c = lax.axis_index("c")
    s = lax.axis_index("s")

    r0 = s * ROWS_PER_TILE
    pltpu.sync_copy(hs_hbm.at[pl.ds(r0, ROWS_PER_TILE)], acc.at[pl.ds(r0, ROWS_PER_TILE)])
    plsc.subcore_barrier()

    def chunk(i, carry):
        pltpu.async_copy(h_hbm.at[cidx.at[i, 0]], rows, gsem).wait()
        pltpu.sync_copy(rows, acc.at[cidx.at[i, 1]], add=True)
        return carry

    def rnd(r, carry):
        pltpu.sync_copy(eidx_hbm.at[c, s, pl.ds(r * ROUND, ROUND)], cidx)
        lax.fori_loop(0, ROUND, chunk, 0)
        return carry

    rounds = jnp.where(c == FAST2, 2, 1)
    lax.fori_loop(0, rounds, rnd, 0)
    plsc.subcore_barrier()
    pltpu.sync_copy(
        acc.at[pl.ds(r0, ROWS_PER_TILE)],
        out_hbm.at[pl.ds(c * NP + r0, ROWS_PER_TILE)],
    )


_sc_call = pl.kernel(
    _sc_body,
    out_type=jax.ShapeDtypeStruct((2 * NP, D), jnp.float32),
    mesh=plsc.VectorSubcoreMesh(core_axis_name="c", subcore_axis_name="s"),
    scratch_types=[
        pltpu.VMEM((ROUND, 2, CHUNK), jnp.int32),
        pltpu.VMEM((CHUNK, D), jnp.float32),
        pltpu.VMEM_SHARED((NP, D), jnp.float32),
        pltpu.SemaphoreType.DMA,
    ],
)


def _combine_body(p_ref, o_ref):
    o_ref[...] = p_ref[0:N, :] + p_ref[NP : NP + N, :]


_combine_call = pl.pallas_call(
    _combine_body,
    out_shape=jax.ShapeDtypeStruct((N, D), jnp.float32),
)


def kernel(n_feat, edge_index, gamma, beta, W):
    edge_index = edge_index.astype(jnp.int32)
    h, hsh = _dense_call(
        n_feat,
        gamma.reshape(1, D),
        beta.reshape(1, D),
        W.T,
    )
    dst = edge_index[0]
    src = edge_index[1]
    pad = E_PAD - E
    src_p = jnp.concatenate([src, jnp.zeros((pad,), jnp.int32)])
    dst_p = jnp.concatenate([dst, jnp.full((pad,), TRASH_ROW, jnp.int32)])
    def blocks(sl, dl, n_chunks):
        s4 = sl.reshape(NS, n_chunks, 1, CHUNK)
        d4 = dl.reshape(NS, n_chunks, 1, CHUNK)
        return jnp.concatenate([s4, d4], axis=2)

    bf = blocks(src_p[:E_FAST], dst_p[:E_FAST], 2 * ROUND)
    bs = blocks(src_p[E_FAST:], dst_p[E_FAST:], ROUND)
    bs = jnp.pad(bs, ((0, 0), (0, ROUND), (0, 0), (0, 0)))
    pair = [None, None]
    pair[FAST2] = bf
    pair[1 - FAST2] = bs
    eidx = jnp.stack(pair, axis=0)
    partials = _sc_call(h, eidx, hsh)
    return _combine_call(partials)
